# initial kernel scaffold (unmeasured)
import jax
import jax.numpy as jnp
from jax import lax
from jax.experimental import pallas as pl
from jax.experimental.pallas import tpu as pltpu

N_DEV = 4
BM = 1024
BK = 1024
BN = 2048
N_TOTAL = 8192
N_TILES = N_TOTAL // BN


def kernel(x, w_mat):
    x = x.astype(jnp.bfloat16)
    m_full, k_shard = x.shape
    k_full, n_total = w_mat.shape

    def body(x_ref, w_ref, out_ref, comm_ref, send_sems, recv_sems):
        n = pl.program_id(0)
        j = pl.program_id(1)
        my = lax.axis_index("i")

        @pl.when(jnp.logical_and(n == 0, j == 0))
        def _():
            barrier = pltpu.get_barrier_semaphore()
            for d in range(N_DEV):
                @pl.when(my != d)
                def _(d=d):
                    pl.semaphore_signal(
                        barrier, inc=1,
                        device_id=(d,), device_id_type=pl.DeviceIdType.MESH,
                    )
            pl.semaphore_wait(barrier, N_DEV - 1)

            comm_ref[my] = x_ref[pl.ds(my * BM, BM), :]

            for d in range(N_DEV):
                @pl.when(my != d)
                def _(d=d):
                    rdma = pltpu.make_async_remote_copy(
                        src_ref=x_ref.at[pl.ds(d * BM, BM), :],
                        dst_ref=comm_ref.at[my],
                        send_sem=send_sems.at[d],
                        recv_sem=recv_sems.at[my],
                        device_id=(d,),
                        device_id_type=pl.DeviceIdType.MESH,
                    )
                    rdma.start()

        @pl.when(jnp.logical_and(n == 0, j != my))
        def _():
            recv = pltpu.make_async_remote_copy(
                src_ref=comm_ref.at[j],
                dst_ref=comm_ref.at[j],
                send_sem=send_sems.at[j],
                recv_sem=recv_sems.at[j],
                device_id=(my,),
                device_id_type=pl.DeviceIdType.MESH,
            )
            recv.wait_recv()

        wb = w_ref[...].astype(jnp.bfloat16)
        partial = jnp.dot(comm_ref[j], wb, preferred_element_type=jnp.float32)

        @pl.when(j == 0)
        def _():
            out_ref[...] = partial

        @pl.when(j != 0)
        def _():
            out_ref[...] += partial

        @pl.when(j == N_DEV - 1)
        def _():
            out_ref[...] = jnp.maximum(out_ref[...], 0.0)

        @pl.when(jnp.logical_and(n == N_TILES - 1, j == N_DEV - 1))
        def _():
            for d in range(N_DEV):
                @pl.when(my != d)
                def _(d=d):
                    send = pltpu.make_async_remote_copy(
                        src_ref=x_ref.at[pl.ds(d * BM, BM), :],
                        dst_ref=comm_ref.at[my],
                        send_sem=send_sems.at[d],
                        recv_sem=recv_sems.at[my],
                        device_id=(d,),
                        device_id_type=pl.DeviceIdType.MESH,
                    )
                    send.wait_send()

    return pl.pallas_call(
        body,
        grid=(N_TILES, N_DEV),
        out_shape=jax.ShapeDtypeStruct((BM, n_total), jnp.float32),
        in_specs=[
            pl.BlockSpec((m_full, k_shard), lambda n, j: (0, 0)),
            pl.BlockSpec((BK, BN), lambda n, j: (j, n)),
        ],
        out_specs=pl.BlockSpec((BM, BN), lambda n, j: (0, n)),
        scratch_shapes=[
            pltpu.VMEM((N_DEV, BM, BK), jnp.bfloat16),
            pltpu.SemaphoreType.DMA((N_DEV,)),
            pltpu.SemaphoreType.DMA((N_DEV,)),
        ],
        compiler_params=pltpu.CompilerParams(
            dimension_semantics=("arbitrary", "arbitrary"),
            collective_id=0,
        ),
    )(x, w_mat)


# baseline (device time: 183098 ns/iter reference)
import jax
import jax.numpy as jnp
from jax import lax
from jax.experimental import pallas as pl
from jax.experimental.pallas import tpu as pltpu

N_DEV = 4
BM = 1024
BK = 1024
BN = 2048
N_TOTAL = 8192
N_TILES = N_TOTAL // BN


def kernel(x, w_mat):
    x = x.astype(jnp.bfloat16)
    m_full, k_shard = x.shape
    k_full, n_total = w_mat.shape

    def body(x_ref, w_ref, out_ref, comm_ref, send_sems, recv_sems):
        n = pl.program_id(0)
        j = pl.program_id(1)
        my = lax.axis_index("i")

        @pl.when(jnp.logical_and(n == 0, j == 0))
        def _():
            barrier = pltpu.get_barrier_semaphore()
            for d in range(N_DEV):
                @pl.when(my != d)
                def _(d=d):
                    pl.semaphore_signal(
                        barrier, inc=1,
                        device_id=(d,), device_id_type=pl.DeviceIdType.MESH,
                    )
            pl.semaphore_wait(barrier, N_DEV - 1)

            comm_ref[my] = x_ref[pl.ds(my * BM, BM), :]

            for d in range(N_DEV):
                @pl.when(my != d)
                def _(d=d):
                    rdma = pltpu.make_async_remote_copy(
                        src_ref=x_ref.at[pl.ds(d * BM, BM), :],
                        dst_ref=comm_ref.at[my],
                        send_sem=send_sems.at[d],
                        recv_sem=recv_sems.at[my],
                        device_id=(d,),
                        device_id_type=pl.DeviceIdType.MESH,
                    )
                    rdma.start()

        @pl.when(jnp.logical_and(n == 0, j != my))
        def _():
            recv = pltpu.make_async_remote_copy(
                src_ref=comm_ref.at[j],
                dst_ref=comm_ref.at[j],
                send_sem=send_sems.at[j],
                recv_sem=recv_sems.at[j],
                device_id=(my,),
                device_id_type=pl.DeviceIdType.MESH,
            )
            recv.wait_recv()

        wb = w_ref[...].astype(jnp.bfloat16)
        partial = jnp.dot(comm_ref[j], wb, preferred_element_type=jnp.float32)

        @pl.when(j == 0)
        def _():
            out_ref[...] = partial

        @pl.when(j != 0)
        def _():
            out_ref[...] += partial

        @pl.when(j == N_DEV - 1)
        def _():
            out_ref[...] = jnp.maximum(out_ref[...], 0.0)

        @pl.when(jnp.logical_and(n == N_TILES - 1, j == N_DEV - 1))
        def _():
            for d in range(N_DEV):
                @pl.when(my != d)
                def _(d=d):
                    send = pltpu.make_async_remote_copy(
                        src_ref=x_ref.at[pl.ds(d * BM, BM), :],
                        dst_ref=comm_ref.at[my],
                        send_sem=send_sems.at[d],
                        recv_sem=recv_sems.at[my],
                        device_id=(d,),
                        device_id_type=pl.DeviceIdType.MESH,
                    )
                    send.wait_send()

    return pl.pallas_call(
        body,
        grid=(N_TILES, N_DEV),
        out_shape=jax.ShapeDtypeStruct((BM, n_total), jnp.float32),
        in_specs=[
            pl.BlockSpec((m_full, k_shard), lambda n, j: (0, 0)),
            pl.BlockSpec((BK, BN), lambda n, j: (j, n)),
        ],
        out_specs=pl.BlockSpec((BM, BN), lambda n, j: (0, n)),
        scratch_shapes=[
            pltpu.VMEM((N_DEV, BM, BK), jnp.bfloat16),
            pltpu.SemaphoreType.DMA((N_DEV,)),
            pltpu.SemaphoreType.DMA((N_DEV,)),
        ],
        compiler_params=pltpu.CompilerParams(
            dimension_semantics=("arbitrary", "arbitrary"),
            collective_id=0,
            vmem_limit_bytes=60 * 1024 * 1024,
        ),
    )(x, w_mat)


# device time: 171618 ns/iter; 1.0669x vs baseline; 1.0669x over previous
import jax
import jax.numpy as jnp
from jax import lax
from jax.experimental import pallas as pl
from jax.experimental.pallas import tpu as pltpu

N_DEV = 4
BM = 1024
BK = 1024
BN = 2048
N_TOTAL = 8192
N_TILES = N_TOTAL // BN


def kernel(x, w_mat):
    x = x.astype(jnp.bfloat16)
    m_full, k_shard = x.shape
    k_full, n_total = w_mat.shape

    my = lax.axis_index("i")
    order = jnp.stack(
        [my, (my + 1) % N_DEV, (my + 3) % N_DEV, (my + 2) % N_DEV]
    ).astype(jnp.int32)

    def body(ord_ref, x_ref, w_ref, out_ref, comm_ref, send_sems, recv_sems):
        n = pl.program_id(0)
        j = pl.program_id(1)
        me = lax.axis_index("i")
        src = ord_ref[j]

        @pl.when(jnp.logical_and(n == 0, j == 0))
        def _():
            barrier = pltpu.get_barrier_semaphore()
            for off in (1, 2, 3):
                pl.semaphore_signal(
                    barrier, inc=1,
                    device_id=((me + off) % N_DEV,),
                    device_id_type=pl.DeviceIdType.MESH,
                )
            pl.semaphore_wait(barrier, N_DEV - 1)

            comm_ref[me] = x_ref[pl.ds(me * BM, BM), :]

            for off in (3, 1, 2):
                d = (me + off) % N_DEV
                rdma = pltpu.make_async_remote_copy(
                    src_ref=x_ref.at[pl.ds(d * BM, BM), :],
                    dst_ref=comm_ref.at[me],
                    send_sem=send_sems.at[d],
                    recv_sem=recv_sems.at[me],
                    device_id=(d,),
                    device_id_type=pl.DeviceIdType.MESH,
                )
                rdma.start()

        @pl.when(jnp.logical_and(n == 0, j > 0))
        def _():
            recv = pltpu.make_async_remote_copy(
                src_ref=comm_ref.at[src],
                dst_ref=comm_ref.at[src],
                send_sem=send_sems.at[src],
                recv_sem=recv_sems.at[src],
                device_id=(me,),
                device_id_type=pl.DeviceIdType.MESH,
            )
            recv.wait_recv()

        partial = lax.dot_general(
            comm_ref[src], w_ref[...],
            (((1,), (0,)), ((), ())),
            preferred_element_type=jnp.float32,
        )

        @pl.when(j == 0)
        def _():
            out_ref[...] = partial

        @pl.when(j != 0)
        def _():
            out_ref[...] += partial

        @pl.when(j == N_DEV - 1)
        def _():
            out_ref[...] = jnp.maximum(out_ref[...], 0.0)

        @pl.when(jnp.logical_and(n == N_TILES - 1, j == N_DEV - 1))
        def _():
            for off in (3, 1, 2):
                d = (me + off) % N_DEV
                send = pltpu.make_async_remote_copy(
                    src_ref=x_ref.at[pl.ds(d * BM, BM), :],
                    dst_ref=comm_ref.at[me],
                    send_sem=send_sems.at[d],
                    recv_sem=recv_sems.at[me],
                    device_id=(d,),
                    device_id_type=pl.DeviceIdType.MESH,
                )
                send.wait_send()

    grid_spec = pltpu.PrefetchScalarGridSpec(
        num_scalar_prefetch=1,
        grid=(N_TILES, N_DEV),
        in_specs=[
            pl.BlockSpec((m_full, k_shard), lambda n, j, o: (0, 0)),
            pl.BlockSpec((BK, BN), lambda n, j, o: (o[j], n)),
        ],
        out_specs=pl.BlockSpec((BM, BN), lambda n, j, o: (0, n)),
        scratch_shapes=[
            pltpu.VMEM((N_DEV, BM, BK), jnp.bfloat16),
            pltpu.SemaphoreType.DMA((N_DEV,)),
            pltpu.SemaphoreType.DMA((N_DEV,)),
        ],
    )
    return pl.pallas_call(
        body,
        grid_spec=grid_spec,
        out_shape=jax.ShapeDtypeStruct((BM, n_total), jnp.float32),
        compiler_params=pltpu.CompilerParams(
            dimension_semantics=("arbitrary", "arbitrary"),
            collective_id=0,
            vmem_limit_bytes=60 * 1024 * 1024,
        ),
    )(order, x, w_mat)


# device time: 153751 ns/iter; 1.1909x vs baseline; 1.1162x over previous
import jax
import jax.numpy as jnp
from jax import lax
from jax.experimental import pallas as pl
from jax.experimental.pallas import tpu as pltpu

N_DEV = 4
BM = 1024
BK = 1024
BN = 1024
N_TOTAL = 8192
N_TILES = N_TOTAL // BN


def kernel(x, w_mat):
    x = x.astype(jnp.bfloat16)
    m_full, k_shard = x.shape
    k_full, n_total = w_mat.shape

    my = lax.axis_index("i")
    order = jnp.stack(
        [my, (my + 1) % N_DEV, (my + 3) % N_DEV, (my + 2) % N_DEV]
    ).astype(jnp.int32)

    def body(ord_ref, x_ref, w_ref, out_ref, comm_ref, acc_ref,
             send_sems, recv_sems):
        jj = pl.program_id(0)
        n = pl.program_id(1)
        me = lax.axis_index("i")
        src = ord_ref[jj]

        @pl.when(jnp.logical_and(jj == 0, n == 0))
        def _():
            barrier = pltpu.get_barrier_semaphore()
            for off in (1, 2, 3):
                pl.semaphore_signal(
                    barrier, inc=1,
                    device_id=((me + off) % N_DEV,),
                    device_id_type=pl.DeviceIdType.MESH,
                )
            pl.semaphore_wait(barrier, N_DEV - 1)

            comm_ref[me] = x_ref[pl.ds(me * BM, BM), :]

            for off in (3, 1, 2):
                d = (me + off) % N_DEV
                rdma = pltpu.make_async_remote_copy(
                    src_ref=x_ref.at[pl.ds(d * BM, BM), :],
                    dst_ref=comm_ref.at[me],
                    send_sem=send_sems.at[d],
                    recv_sem=recv_sems.at[me],
                    device_id=(d,),
                    device_id_type=pl.DeviceIdType.MESH,
                )
                rdma.start()

        @pl.when(jnp.logical_and(jj > 0, n == 0))
        def _():
            recv = pltpu.make_async_remote_copy(
                src_ref=comm_ref.at[src],
                dst_ref=comm_ref.at[src],
                send_sem=send_sems.at[src],
                recv_sem=recv_sems.at[src],
                device_id=(me,),
                device_id_type=pl.DeviceIdType.MESH,
            )
            recv.wait_recv()

        partial = lax.dot_general(
            comm_ref[src], w_ref[...],
            (((1,), (0,)), ((), ())),
            preferred_element_type=jnp.float32,
        )
        nsl = pl.ds(n * BN, BN)

        @pl.when(jj == 0)
        def _():
            acc_ref[:, nsl] = partial.astype(jnp.bfloat16)

        @pl.when(jnp.logical_and(jj > 0, jj < N_DEV - 1))
        def _():
            acc_ref[:, nsl] = (
                acc_ref[:, nsl].astype(jnp.float32) + partial
            ).astype(jnp.bfloat16)

        @pl.when(jj == N_DEV - 1)
        def _():
            out_ref[...] = jnp.maximum(
                acc_ref[:, nsl].astype(jnp.float32) + partial, 0.0
            )

        @pl.when(jnp.logical_and(jj == N_DEV - 1, n == N_TILES - 1))
        def _():
            for off in (3, 1, 2):
                d = (me + off) % N_DEV
                send = pltpu.make_async_remote_copy(
                    src_ref=x_ref.at[pl.ds(d * BM, BM), :],
                    dst_ref=comm_ref.at[me],
                    send_sem=send_sems.at[d],
                    recv_sem=recv_sems.at[me],
                    device_id=(d,),
                    device_id_type=pl.DeviceIdType.MESH,
                )
                send.wait_send()

    grid_spec = pltpu.PrefetchScalarGridSpec(
        num_scalar_prefetch=1,
        grid=(N_DEV, N_TILES),
        in_specs=[
            pl.BlockSpec((m_full, k_shard), lambda jj, n, o: (0, 0)),
            pl.BlockSpec((BK, BN), lambda jj, n, o: (o[jj], n)),
        ],
        out_specs=pl.BlockSpec((BM, BN), lambda jj, n, o: (0, n)),
        scratch_shapes=[
            pltpu.VMEM((N_DEV, BM, BK), jnp.bfloat16),
            pltpu.VMEM((BM, N_TOTAL), jnp.bfloat16),
            pltpu.SemaphoreType.DMA((N_DEV,)),
            pltpu.SemaphoreType.DMA((N_DEV,)),
        ],
    )
    return pl.pallas_call(
        body,
        grid_spec=grid_spec,
        out_shape=jax.ShapeDtypeStruct((BM, n_total), jnp.float32),
        compiler_params=pltpu.CompilerParams(
            dimension_semantics=("arbitrary", "arbitrary"),
            collective_id=0,
            vmem_limit_bytes=60 * 1024 * 1024,
        ),
    )(order, x, w_mat)
